# baseline (speedup 1.0000x reference)
import jax
import jax.numpy as jnp
from jax import lax
from jax.experimental import pallas as pl
from jax.experimental.pallas import tpu as pltpu
from jax.experimental.pallas import tpu_sc as plsc

B, S, D = 4, 4096, 2048
NTOK = B * S
NWORKERS = 32
TOK_PER_W = NTOK // NWORKERS
CHUNK = 16
NCHUNK = TOK_PER_W // CHUNK
NSEG = 16
JSEG = 8
EPS = 1e-5


_GDN = lax.GatherDimensionNumbers(
    offset_dims=(), collapsed_slice_dims=(0,), start_index_map=(0,))


def _lane_shuffle(v, idx):
    return lax.gather(v, idx[:, None], dimension_numbers=_GDN, slice_sizes=(1,),
                      mode=lax.GatherScatterMode.PROMISE_IN_BOUNDS)


def _lane_sum(v, lanes):
    for sh in (8, 4, 2, 1):
        v = v + _lane_shuffle(v, lanes ^ sh)
    return v


def _rsqrt_newton(v):
    bits = lax.bitcast_convert_type(v, jnp.int32)
    y = lax.bitcast_convert_type(jnp.int32(0x5F3759DF) - (bits >> 1), jnp.float32)
    for _ in range(3):
        y = y * (1.5 - 0.5 * v * y * y)
    return y


def _sc_body(h_hbm, idx_hbm, w0_hbm, b0_hbm, w1_hbm, b1_hbm, out_hbm,
             buf, w0_v, b0_v, w1_v, b1_v, idx_v, mu_sp, rstd_sp, sel_sp,
             in_sem, out_sem):
    ncores = plsc.get_sparse_core_info().num_cores
    wid = lax.axis_index("s") * ncores + lax.axis_index("c")
    tok0 = wid * TOK_PER_W
    chunk0 = wid * NCHUNK

    pltpu.sync_copy(w0_hbm, w0_v)
    pltpu.sync_copy(b0_hbm, b0_v)
    pltpu.sync_copy(w1_hbm, w1_v)
    pltpu.sync_copy(b1_hbm, b1_v)
    pltpu.sync_copy(idx_hbm.at[pl.ds(tok0, TOK_PER_W)], idx_v)

    lanes = lax.iota(jnp.int32, 16)
    zero16 = jnp.zeros((16,), jnp.float32)

    pltpu.async_copy(h_hbm.at[chunk0], buf.at[0], in_sem)
    pltpu.async_copy(h_hbm.at[chunk0 + 1], buf.at[1], in_sem)

    def chunk_body(c, _):
        sl = lax.rem(c, 3)
        pltpu.make_async_copy(h_hbm.at[chunk0 + c], buf.at[sl], in_sem).wait()

        tv_v = idx_v[pl.ds(c * CHUNK, 16)]

        @plsc.parallel_loop(0, CHUNK, 1, unroll=2)
        def _p1(t):
            ax = [zero16] * 4
            aq = [zero16] * 4
            for j in range(NSEG * JSEG):
                x = buf[sl, t, j // JSEG, pl.ds((j % JSEG) * 16, 16)]
                k = j & 3
                ax[k] = ax[k] + x
                aq[k] = aq[k] + x * x
            mu = _lane_sum((ax[0] + ax[1]) + (ax[2] + ax[3]), lanes) * (1.0 / D)
            var = _lane_sum((aq[0] + aq[1]) + (aq[2] + aq[3]), lanes) * (1.0 / D) - mu * mu
            mu_sp[t] = mu
            rstd_sp[t] = _rsqrt_newton(var + EPS)
            sel_sp[t] = _lane_shuffle(tv_v, jnp.full((16,), t, jnp.int32))

        @plsc.parallel_loop(0, NSEG, 1, unroll=1)
        def _p2(s):
            w0r = [w0_v[s, pl.ds(k * 16, 16)] for k in range(JSEG)]
            w1r = [w1_v[s, pl.ds(k * 16, 16)] for k in range(JSEG)]
            b0r = [b0_v[s, pl.ds(k * 16, 16)] for k in range(JSEG)]
            b1r = [b1_v[s, pl.ds(k * 16, 16)] for k in range(JSEG)]
            for t in range(CHUNK):
                mu_s = mu_sp[t]
                rstd_s = rstd_sp[t]
                sel1 = sel_sp[t] >= 0.5
                for k in range(JSEG):
                    x = buf[sl, t, s, pl.ds(k * 16, 16)]
                    wj = jnp.where(sel1, w1r[k], w0r[k])
                    bj = jnp.where(sel1, b1r[k], b0r[k])
                    buf[sl, t, s, pl.ds(k * 16, 16)] = (x - mu_s) * rstd_s * wj + bj

        pltpu.async_copy(buf.at[sl], out_hbm.at[chunk0 + c], out_sem)
        @pl.when(c >= 1)
        def _():
            pltpu.make_async_copy(buf.at[lax.rem(c - 1, 3)],
                                  out_hbm.at[chunk0 + c - 1], out_sem).wait()

        @pl.when(c + 2 < NCHUNK)
        def _():
            pltpu.async_copy(h_hbm.at[chunk0 + c + 2],
                             buf.at[lax.rem(c + 2, 3)], in_sem)

        return 0

    lax.fori_loop(0, NCHUNK, chunk_body, 0)
    pltpu.make_async_copy(buf.at[lax.rem(NCHUNK - 1, 3)],
                          out_hbm.at[chunk0 + NCHUNK - 1], out_sem).wait()


@jax.jit
def kernel(hidden_states, multiway_indices, ln0_w, ln0_b, ln1_w, ln1_b):
    h4 = hidden_states.reshape(NTOK // CHUNK, CHUNK, NSEG, JSEG * 16)
    idx_flat = multiway_indices.reshape(-1).astype(jnp.float32)

    mesh = plsc.VectorSubcoreMesh(core_axis_name="c", subcore_axis_name="s")
    run = pl.kernel(
        _sc_body,
        out_type=jax.ShapeDtypeStruct((NTOK // CHUNK, CHUNK, NSEG, JSEG * 16),
                                      jnp.float32),
        mesh=mesh,
        compiler_params=pltpu.CompilerParams(needs_layout_passes=False),
        scratch_types=[
            pltpu.VMEM((3, CHUNK, NSEG, JSEG * 16), jnp.float32),
            pltpu.VMEM((NSEG, JSEG * 16), jnp.float32),
            pltpu.VMEM((NSEG, JSEG * 16), jnp.float32),
            pltpu.VMEM((NSEG, JSEG * 16), jnp.float32),
            pltpu.VMEM((NSEG, JSEG * 16), jnp.float32),
            pltpu.VMEM((TOK_PER_W,), jnp.float32),
            pltpu.VMEM((CHUNK, 16), jnp.float32),
            pltpu.VMEM((CHUNK, 16), jnp.float32),
            pltpu.VMEM((CHUNK, 16), jnp.float32),
            pltpu.SemaphoreType.DMA,
            pltpu.SemaphoreType.DMA,
        ],
    )
    out = run(h4, idx_flat,
              ln0_w.reshape(NSEG, JSEG * 16), ln0_b.reshape(NSEG, JSEG * 16),
              ln1_w.reshape(NSEG, JSEG * 16), ln1_b.reshape(NSEG, JSEG * 16))
    return out.reshape(B, S, D)


# speedup vs baseline: 2.1380x; 2.1380x over previous
import jax
import jax.numpy as jnp
from jax import lax
from jax.experimental import pallas as pl
from jax.experimental.pallas import tpu as pltpu
from jax.experimental.pallas import tpu_sc as plsc

B, S, D = 4, 4096, 2048
NTOK = B * S
NWORKERS = 32
TOK_PER_W = NTOK // NWORKERS
CHUNK = 16
NCHUNK = TOK_PER_W // CHUNK
NSEG = 16
JSEG = 8
EPS = 1e-5


_GDN = lax.GatherDimensionNumbers(
    offset_dims=(), collapsed_slice_dims=(0,), start_index_map=(0,))


def _lane_shuffle(v, idx):
    return lax.gather(v, idx[:, None], dimension_numbers=_GDN, slice_sizes=(1,),
                      mode=lax.GatherScatterMode.PROMISE_IN_BOUNDS)


def _lane_sum(v, lanes):
    for sh in (8, 4, 2, 1):
        v = v + _lane_shuffle(v, lanes ^ sh)
    return v


def _rsqrt_newton(v):
    bits = lax.bitcast_convert_type(v, jnp.int32)
    y = lax.bitcast_convert_type(jnp.int32(0x5F3759DF) - (bits >> 1), jnp.float32)
    for _ in range(3):
        y = y * (1.5 - 0.5 * v * y * y)
    return y


def _sc_body(h_hbm, idx_hbm, w0_hbm, b0_hbm, w1_hbm, b1_hbm, out_hbm,
             buf, w0_v, b0_v, w1_v, b1_v, idx_v, mu_sp, rstd_sp, sel_sp,
             in_sem, out_sem):
    ncores = plsc.get_sparse_core_info().num_cores
    wid = lax.axis_index("s") * ncores + lax.axis_index("c")
    tok0 = wid * TOK_PER_W
    chunk0 = wid * NCHUNK

    pltpu.sync_copy(w0_hbm, w0_v)
    pltpu.sync_copy(b0_hbm, b0_v)
    pltpu.sync_copy(w1_hbm, w1_v)
    pltpu.sync_copy(b1_hbm, b1_v)
    pltpu.sync_copy(idx_hbm.at[pl.ds(tok0, TOK_PER_W)], idx_v)

    lanes = lax.iota(jnp.int32, 16)
    zero16 = jnp.zeros((16,), jnp.float32)

    pltpu.async_copy(h_hbm.at[pl.ds(tok0, CHUNK)], buf.at[0], in_sem)
    pltpu.async_copy(h_hbm.at[pl.ds(tok0 + CHUNK, CHUNK)], buf.at[1], in_sem)

    def chunk_body(c, _):
        sl = lax.rem(c, 3)
        row = tok0 + c * CHUNK
        pltpu.make_async_copy(h_hbm.at[pl.ds(row, CHUNK)], buf.at[sl], in_sem).wait()

        tv_v = idx_v[pl.ds(c * CHUNK, 16)]

        @plsc.parallel_loop(0, CHUNK, 1, unroll=2)
        def _p1(t):
            ax = [zero16] * 4
            aq = [zero16] * 4
            for j in range(NSEG * JSEG):
                x = buf[sl, t, pl.ds(j * 16, 16)]
                k = j & 3
                ax[k] = ax[k] + x
                aq[k] = aq[k] + x * x
            mu = _lane_sum((ax[0] + ax[1]) + (ax[2] + ax[3]), lanes) * (1.0 / D)
            var = _lane_sum((aq[0] + aq[1]) + (aq[2] + aq[3]), lanes) * (1.0 / D) - mu * mu
            mu_sp[t] = mu
            rstd_sp[t] = _rsqrt_newton(var + EPS)
            sel_sp[t] = _lane_shuffle(tv_v, jnp.full((16,), t, jnp.int32))

        @plsc.parallel_loop(0, NSEG, 1, unroll=1)
        def _p2(s):
            w0r = [w0_v[s, pl.ds(k * 16, 16)] for k in range(JSEG)]
            w1r = [w1_v[s, pl.ds(k * 16, 16)] for k in range(JSEG)]
            b0r = [b0_v[s, pl.ds(k * 16, 16)] for k in range(JSEG)]
            b1r = [b1_v[s, pl.ds(k * 16, 16)] for k in range(JSEG)]
            for t in range(CHUNK):
                mu_s = mu_sp[t]
                rstd_s = rstd_sp[t]
                sel1 = sel_sp[t] >= 0.5
                for k in range(JSEG):
                    x = buf[sl, t, pl.ds(s * 128 + k * 16, 16)]
                    wj = jnp.where(sel1, w1r[k], w0r[k])
                    bj = jnp.where(sel1, b1r[k], b0r[k])
                    buf[sl, t, pl.ds(s * 128 + k * 16, 16)] = (x - mu_s) * rstd_s * wj + bj

        pltpu.async_copy(buf.at[sl], out_hbm.at[pl.ds(row, CHUNK)], out_sem)
        @pl.when(c >= 1)
        def _():
            pltpu.make_async_copy(buf.at[lax.rem(c - 1, 3)],
                                  out_hbm.at[pl.ds(row - CHUNK, CHUNK)],
                                  out_sem).wait()

        @pl.when(c + 2 < NCHUNK)
        def _():
            pltpu.async_copy(h_hbm.at[pl.ds(row + 2 * CHUNK, CHUNK)],
                             buf.at[lax.rem(c + 2, 3)], in_sem)

        return 0

    lax.fori_loop(0, NCHUNK, chunk_body, 0)
    pltpu.make_async_copy(buf.at[lax.rem(NCHUNK - 1, 3)],
                          out_hbm.at[pl.ds(tok0 + (NCHUNK - 1) * CHUNK, CHUNK)],
                          out_sem).wait()


@jax.jit
def kernel(hidden_states, multiway_indices, ln0_w, ln0_b, ln1_w, ln1_b):
    h2 = hidden_states.reshape(NTOK, D)
    idx_flat = multiway_indices.reshape(-1).astype(jnp.float32)

    mesh = plsc.VectorSubcoreMesh(core_axis_name="c", subcore_axis_name="s")
    run = pl.kernel(
        _sc_body,
        out_type=jax.ShapeDtypeStruct((NTOK, D), jnp.float32),
        mesh=mesh,
        compiler_params=pltpu.CompilerParams(needs_layout_passes=False),
        scratch_types=[
            pltpu.VMEM((3, CHUNK, D), jnp.float32),
            pltpu.VMEM((NSEG, JSEG * 16), jnp.float32),
            pltpu.VMEM((NSEG, JSEG * 16), jnp.float32),
            pltpu.VMEM((NSEG, JSEG * 16), jnp.float32),
            pltpu.VMEM((NSEG, JSEG * 16), jnp.float32),
            pltpu.VMEM((TOK_PER_W,), jnp.float32),
            pltpu.VMEM((CHUNK, 16), jnp.float32),
            pltpu.VMEM((CHUNK, 16), jnp.float32),
            pltpu.VMEM((CHUNK, 16), jnp.float32),
            pltpu.SemaphoreType.DMA,
            pltpu.SemaphoreType.DMA,
        ],
    )
    out = run(h2, idx_flat,
              ln0_w.reshape(NSEG, JSEG * 16), ln0_b.reshape(NSEG, JSEG * 16),
              ln1_w.reshape(NSEG, JSEG * 16), ln1_b.reshape(NSEG, JSEG * 16))
    return out.reshape(B, S, D)


# speedup vs baseline: 2.1843x; 1.0217x over previous
import jax
import jax.numpy as jnp
from jax import lax
from jax.experimental import pallas as pl
from jax.experimental.pallas import tpu as pltpu
from jax.experimental.pallas import tpu_sc as plsc

B, S, D = 4, 4096, 2048
NTOK = B * S
NWORKERS = 32
TOK_PER_W = NTOK // NWORKERS
CHUNK = 16
NCHUNK = TOK_PER_W // CHUNK
NSEG = 16
JSEG = 8
EPS = 1e-5


_GDN = lax.GatherDimensionNumbers(
    offset_dims=(), collapsed_slice_dims=(0,), start_index_map=(0,))


def _lane_shuffle(v, idx):
    return lax.gather(v, idx[:, None], dimension_numbers=_GDN, slice_sizes=(1,),
                      mode=lax.GatherScatterMode.PROMISE_IN_BOUNDS)


def _lane_sum(v, lanes):
    for sh in (8, 4, 2, 1):
        v = v + _lane_shuffle(v, lanes ^ sh)
    return v


def _rsqrt_newton(v):
    bits = lax.bitcast_convert_type(v, jnp.int32)
    y = lax.bitcast_convert_type(jnp.int32(0x5F3759DF) - (bits >> 1), jnp.float32)
    for _ in range(3):
        y = y * (1.5 - 0.5 * v * y * y)
    return y


def _sc_body(h_hbm, idx_hbm, w0_hbm, b0_hbm, w1_hbm, b1_hbm, out_hbm,
             buf, w0_v, b0_v, w1_v, b1_v, idx_v, mu_sp, rstd_sp, sel_sp,
             in_sem, out_sem):
    ncores = plsc.get_sparse_core_info().num_cores
    wid = lax.axis_index("s") * ncores + lax.axis_index("c")
    tok0 = wid * TOK_PER_W
    chunk0 = wid * NCHUNK

    pltpu.sync_copy(w0_hbm, w0_v)
    pltpu.sync_copy(b0_hbm, b0_v)
    pltpu.sync_copy(w1_hbm, w1_v)
    pltpu.sync_copy(b1_hbm, b1_v)
    pltpu.sync_copy(idx_hbm.at[pl.ds(tok0, TOK_PER_W)], idx_v)

    lanes = lax.iota(jnp.int32, 16)
    zero16 = jnp.zeros((16,), jnp.float32)

    pltpu.async_copy(h_hbm.at[pl.ds(tok0, CHUNK)], buf.at[0], in_sem)
    pltpu.async_copy(h_hbm.at[pl.ds(tok0 + CHUNK, CHUNK)], buf.at[1], in_sem)

    def chunk_body(c, _):
        sl = lax.rem(c, 3)
        row = tok0 + c * CHUNK
        pltpu.make_async_copy(h_hbm.at[pl.ds(row, CHUNK)], buf.at[sl], in_sem).wait()

        tv_v = idx_v[pl.ds(c * CHUNK, 16)]

        @plsc.parallel_loop(0, CHUNK, 1, unroll=4)
        def _p1(t):
            ax = [zero16] * 4
            aq = [zero16] * 4
            for j in range(NSEG * JSEG):
                x = buf[sl, t, pl.ds(j * 16, 16)]
                k = j & 3
                ax[k] = ax[k] + x
                aq[k] = aq[k] + x * x
            mu = _lane_sum((ax[0] + ax[1]) + (ax[2] + ax[3]), lanes) * (1.0 / D)
            var = _lane_sum((aq[0] + aq[1]) + (aq[2] + aq[3]), lanes) * (1.0 / D) - mu * mu
            mu_sp[t] = mu
            rstd_sp[t] = _rsqrt_newton(var + EPS)
            sel_sp[t] = _lane_shuffle(tv_v, jnp.full((16,), t, jnp.int32))

        @plsc.parallel_loop(0, NSEG, 1, unroll=1)
        def _p2(s):
            w0r = [w0_v[s, pl.ds(k * 16, 16)] for k in range(JSEG)]
            w1r = [w1_v[s, pl.ds(k * 16, 16)] for k in range(JSEG)]
            b0r = [b0_v[s, pl.ds(k * 16, 16)] for k in range(JSEG)]
            b1r = [b1_v[s, pl.ds(k * 16, 16)] for k in range(JSEG)]
            for t in range(CHUNK):
                mu_s = mu_sp[t]
                rstd_s = rstd_sp[t]
                sel1 = sel_sp[t] >= 0.5
                for k in range(JSEG):
                    x = buf[sl, t, pl.ds(s * 128 + k * 16, 16)]
                    wj = jnp.where(sel1, w1r[k], w0r[k])
                    bj = jnp.where(sel1, b1r[k], b0r[k])
                    buf[sl, t, pl.ds(s * 128 + k * 16, 16)] = (x - mu_s) * rstd_s * wj + bj

        pltpu.async_copy(buf.at[sl], out_hbm.at[pl.ds(row, CHUNK)], out_sem)
        @pl.when(c >= 1)
        def _():
            pltpu.make_async_copy(buf.at[lax.rem(c - 1, 3)],
                                  out_hbm.at[pl.ds(row - CHUNK, CHUNK)],
                                  out_sem).wait()

        @pl.when(c + 2 < NCHUNK)
        def _():
            pltpu.async_copy(h_hbm.at[pl.ds(row + 2 * CHUNK, CHUNK)],
                             buf.at[lax.rem(c + 2, 3)], in_sem)

        return 0

    lax.fori_loop(0, NCHUNK, chunk_body, 0)
    pltpu.make_async_copy(buf.at[lax.rem(NCHUNK - 1, 3)],
                          out_hbm.at[pl.ds(tok0 + (NCHUNK - 1) * CHUNK, CHUNK)],
                          out_sem).wait()


@jax.jit
def kernel(hidden_states, multiway_indices, ln0_w, ln0_b, ln1_w, ln1_b):
    h2 = hidden_states.reshape(NTOK, D)
    idx_flat = multiway_indices.reshape(-1).astype(jnp.float32)

    mesh = plsc.VectorSubcoreMesh(core_axis_name="c", subcore_axis_name="s")
    run = pl.kernel(
        _sc_body,
        out_type=jax.ShapeDtypeStruct((NTOK, D), jnp.float32),
        mesh=mesh,
        compiler_params=pltpu.CompilerParams(needs_layout_passes=False),
        scratch_types=[
            pltpu.VMEM((3, CHUNK, D), jnp.float32),
            pltpu.VMEM((NSEG, JSEG * 16), jnp.float32),
            pltpu.VMEM((NSEG, JSEG * 16), jnp.float32),
            pltpu.VMEM((NSEG, JSEG * 16), jnp.float32),
            pltpu.VMEM((NSEG, JSEG * 16), jnp.float32),
            pltpu.VMEM((TOK_PER_W,), jnp.float32),
            pltpu.VMEM((CHUNK, 16), jnp.float32),
            pltpu.VMEM((CHUNK, 16), jnp.float32),
            pltpu.VMEM((CHUNK, 16), jnp.float32),
            pltpu.SemaphoreType.DMA,
            pltpu.SemaphoreType.DMA,
        ],
    )
    out = run(h2, idx_flat,
              ln0_w.reshape(NSEG, JSEG * 16), ln0_b.reshape(NSEG, JSEG * 16),
              ln1_w.reshape(NSEG, JSEG * 16), ln1_b.reshape(NSEG, JSEG * 16))
    return out.reshape(B, S, D)
